# Initial kernel scaffold; baseline (speedup 1.0000x reference)
import jax
import jax.numpy as jnp
from jax.experimental import pallas as pl


def kernel(src, idx):
    raise NotImplementedError("write your pallas kernel here")


# speedup vs baseline: 48.4817x; 48.4817x over previous
import dataclasses
import functools

import jax
import jax.numpy as jnp
from jax import lax
from jax.experimental import pallas as pl
from jax.experimental.pallas import tpu as pltpu
from jax.experimental.pallas import tpu_sc as plsc

NUM_BINS = 1024
B = 4096
L = 4096

NC = 2
NS = 16
LANES = 16
NW = NC * NS
ROWS_PER_W = B // NW
RGROUPS = ROWS_PER_W // LANES
CHUNK = 512
NCHUNK = L // CHUNK


def kernel(src, idx):
    mesh = plsc.VectorSubcoreMesh(core_axis_name="c", subcore_axis_name="s")
    cp = pltpu.CompilerParams()
    if "needs_layout_passes" in pltpu.CompilerParams.__dataclass_fields__:
        cp = dataclasses.replace(cp, needs_layout_passes=False)

    @functools.partial(
        pl.kernel,
        compiler_params=cp,
        out_type=jax.ShapeDtypeStruct((B, NUM_BINS), jnp.float32),
        mesh=mesh,
        scratch_types=[
            pltpu.VMEM((LANES, CHUNK), jnp.float32),
            pltpu.VMEM((LANES, CHUNK), jnp.int32),
            pltpu.VMEM((LANES, NUM_BINS), jnp.float32),
        ],
    )
    def run(src_hbm, idx_hbm, out_hbm, sblk, iblk, acc):
        wid = lax.axis_index("s") * NC + lax.axis_index("c")
        lane = jnp.arange(LANES, dtype=jnp.int32)
        neg_inf = jnp.full((LANES,), -jnp.inf, dtype=jnp.float32)
        zero = jnp.zeros((LANES,), dtype=jnp.float32)

        @pl.loop(0, RGROUPS)
        def _(g):
            r0 = wid * ROWS_PER_W + g * LANES

            @pl.loop(0, LANES)
            def _(l):
                @pl.loop(0, NUM_BINS, step=LANES)
                def _(b):
                    acc[l, pl.ds(b, LANES)] = neg_inf

            @pl.loop(0, NCHUNK)
            def _(ci):
                c0 = ci * CHUNK
                pltpu.sync_copy(
                    src_hbm.at[pl.ds(r0, LANES), pl.ds(c0, CHUNK)], sblk)
                pltpu.sync_copy(
                    idx_hbm.at[pl.ds(r0, LANES), pl.ds(c0, CHUNK)], iblk)

                @pl.loop(0, CHUNK)
                def _(j):
                    jv = jnp.full((LANES,), j, dtype=jnp.int32)
                    gi = plsc.load_gather(iblk, [lane, jv])
                    gv = plsc.load_gather(sblk, [lane, jv])
                    old = plsc.load_gather(acc, [lane, gi])
                    plsc.store_scatter(acc, [lane, gi], jnp.maximum(old, gv))

            @pl.loop(0, LANES)
            def _(l):
                @pl.loop(0, NUM_BINS, step=LANES)
                def _(b):
                    v = acc[l, pl.ds(b, LANES)]
                    acc[l, pl.ds(b, LANES)] = jnp.where(
                        v == neg_inf, zero, v)

            pltpu.sync_copy(acc, out_hbm.at[pl.ds(r0, LANES), :])

    return run(src, idx)
